# Initial kernel scaffold; baseline (speedup 1.0000x reference)
import jax
import jax.numpy as jnp
from jax.experimental import pallas as pl


def kernel(x, table):
    raise NotImplementedError("write your pallas kernel here")


# speedup vs baseline: 3.8853x; 3.8853x over previous
import functools

import jax
import jax.numpy as jnp
from jax import lax
from jax.experimental import pallas as pl
from jax.experimental.pallas import tpu as pltpu
from jax.experimental.pallas import tpu_sc as plsc

D = 64
IDX_ROW = 128
CHUNK = 512


@functools.cache
def _make_gather(B):
    info = plsc.get_sparse_core_info()
    nw = info.num_cores * info.num_subcores
    assert B % (nw * CHUNK) == 0
    b_per_w = B // nw
    n_chunks = b_per_w // CHUNK
    k = CHUNK // IDX_ROW

    mesh = plsc.VectorSubcoreMesh(core_axis_name="c", subcore_axis_name="s")

    @functools.partial(
        pl.kernel,
        mesh=mesh,
        out_type=jax.ShapeDtypeStruct((B, D), jnp.float32),
        scratch_types=[
            pltpu.VMEM((k, IDX_ROW), jnp.int32),
            pltpu.VMEM((CHUNK, D), jnp.float32),
            pltpu.SemaphoreType.DMA,
        ],
        compiler_params=pltpu.CompilerParams(use_tc_tiling_on_sc=False),
    )
    def emb(x_hbm, table_hbm, out_hbm, idx_v, rows_v, sem):
        wid = lax.axis_index("s") * info.num_cores + lax.axis_index("c")
        row0 = wid * (b_per_w // IDX_ROW)

        def chunk_body(g, carry):
            pltpu.sync_copy(x_hbm.at[pl.ds(row0 + g * k, k)], idx_v)
            copies = [
                pltpu.async_copy(
                    table_hbm.at[idx_v.at[j]],
                    rows_v.at[pl.ds(j * IDX_ROW, IDX_ROW)],
                    sem,
                )
                for j in range(k)
            ]
            for c in copies:
                c.wait()
            pltpu.sync_copy(
                rows_v, out_hbm.at[pl.ds(wid * b_per_w + g * CHUNK, CHUNK)]
            )
            return carry

        lax.fori_loop(0, n_chunks, chunk_body, 0)

    return emb


def kernel(x, table):
    orig_shape = x.shape
    B = x.size
    x2d = x.reshape(B // IDX_ROW, IDX_ROW).astype(jnp.int32)
    out = _make_gather(B)(x2d, table)
    return out.reshape(*orig_shape, D)


# speedup vs baseline: 4.1729x; 1.0740x over previous
import functools

import jax
import jax.numpy as jnp
from jax import lax
from jax.experimental import pallas as pl
from jax.experimental.pallas import tpu as pltpu
from jax.experimental.pallas import tpu_sc as plsc

D = 64
IDX_ROW = 128
CHUNK = 256
NBUF = 4
K = CHUNK // IDX_ROW


@functools.cache
def _make_gather(B):
    info = plsc.get_sparse_core_info()
    nw = info.num_cores * info.num_subcores
    assert B % (nw * CHUNK * NBUF) == 0
    b_per_w = B // nw
    n_chunks = b_per_w // CHUNK
    n_groups = n_chunks // NBUF
    n_idx_rows = b_per_w // IDX_ROW

    mesh = plsc.VectorSubcoreMesh(core_axis_name="c", subcore_axis_name="s")

    @functools.partial(
        pl.kernel,
        mesh=mesh,
        out_type=jax.ShapeDtypeStruct((B, D), jnp.float32),
        scratch_types=[
            pltpu.VMEM((n_idx_rows, IDX_ROW), jnp.int32),
            pltpu.VMEM((NBUF, CHUNK, D), jnp.float32),
        ]
        + [pltpu.SemaphoreType.DMA] * (2 * NBUF),
        compiler_params=pltpu.CompilerParams(use_tc_tiling_on_sc=False),
    )
    def emb(x_hbm, table_hbm, out_hbm, idx_all, rows, *sems):
        sem_g, sem_w = sems[:NBUF], sems[NBUF:]
        wid = lax.axis_index("s") * info.num_cores + lax.axis_index("c")
        row0 = wid * n_idx_rows
        out0 = wid * b_per_w

        pltpu.sync_copy(x_hbm.at[pl.ds(row0, n_idx_rows)], idx_all)

        def fire_gather(c, b):
            for j in range(K):
                pltpu.async_copy(
                    table_hbm.at[idx_all.at[c * K + j]],
                    rows.at[b].at[pl.ds(j * IDX_ROW, IDX_ROW)],
                    sem_g[b],
                )

        def wait_gather(b):
            pltpu.make_async_copy(
                table_hbm.at[pl.ds(0, CHUNK)], rows.at[b], sem_g[b]
            ).wait()

        def fire_write(c, b):
            pltpu.async_copy(
                rows.at[b], out_hbm.at[pl.ds(out0 + c * CHUNK, CHUNK)], sem_w[b]
            )

        def wait_write(c, b):
            pltpu.make_async_copy(
                rows.at[b], out_hbm.at[pl.ds(out0 + c * CHUNK, CHUNK)], sem_w[b]
            ).wait()

        fire_gather(0, 0)
        fire_gather(1, 1)

        def group(t, carry):
            for b in range(NBUF):
                c = t * NBUF + b
                wait_gather(b)
                fire_write(c, b)
                s2 = (b + 2) % NBUF

                @pl.when(c + 2 < n_chunks)
                def _():
                    @pl.when(c >= 2)
                    def _():
                        wait_write(c - 2, s2)

                    fire_gather(c + 2, s2)

            return carry

        lax.fori_loop(0, n_groups, group, 0)

        for b in range(NBUF):
            wait_write(n_chunks - NBUF + b, b)

    return emb


def kernel(x, table):
    orig_shape = x.shape
    B = x.size
    x2d = x.reshape(B // IDX_ROW, IDX_ROW).astype(jnp.int32)
    out = _make_gather(B)(x2d, table)
    return out.reshape(*orig_shape, D)
